# baseline (speedup 1.0000x reference)
import functools

import jax
import jax.numpy as jnp
import numpy as np
from jax import lax
from jax.experimental import pallas as pl
from jax.experimental.pallas import tpu as pltpu
from jax.experimental.pallas import tpu_sc as plsc

_FIELDS = 26
_EMBED = 16
_NPAIRS = (_FIELDS * (_FIELDS - 1)) // 2
_IU_R, _IU_C = np.triu_indices(_FIELDS, k=1)

_NC = 2
_NS = 16
_NW = _NC * _NS
_G = 4

_ROW = _FIELDS * _EMBED
_OROW = _NPAIRS * _EMBED


def _pair_products(rows_ref, out_ref, gb):
    rbase = gb * _FIELDS
    for p in range(_NPAIRS):
        i = int(_IU_R[p])
        j = int(_IU_C[p])
        a = rows_ref[rbase + i, j, :]
        b = rows_ref[rbase + j, i, :]
        out_ref[gb, p, :] = a * b


def _sc_body(nb, ng, x_hbm, v_hbm, out_hbm,
             idx0, idx1, rows0, rows1, outv0, outv1,
             gsem0, gsem1, osem0, osem1):
    idx = (idx0, idx1)
    rows = (rows0, rows1)
    outv = (outv0, outv1)
    gsem = (gsem0, gsem1)
    osem = (osem0, osem1)

    wid = lax.axis_index("s") * _NC + lax.axis_index("c")
    base = wid * nb

    def start_gather(g, buf):
        pltpu.sync_copy(x_hbm.at[pl.ds((base + g * _G) * _FIELDS, _G * _FIELDS)],
                        idx[buf])
        pltpu.make_async_copy(v_hbm.at[idx[buf]], rows[buf], gsem[buf]).start()

    def wait_gather(buf):
        pltpu.make_async_copy(v_hbm.at[idx[buf]], rows[buf], gsem[buf]).wait()

    def start_scatter(g, buf):
        pltpu.make_async_copy(
            outv[buf],
            out_hbm.at[pl.ds(base + g * _G, _G)],
            osem[buf]).start()

    def wait_scatter(g, buf):
        pltpu.make_async_copy(
            outv[buf],
            out_hbm.at[pl.ds(base + g * _G, _G)],
            osem[buf]).wait()

    start_gather(0, 0)

    def outer(gg, carry):
        for b in (0, 1):
            g = gg * 2 + b

            @pl.when(g + 1 < ng)
            def _():
                start_gather(g + 1, (b + 1) % 2)

            wait_gather(b)

            @pl.when(g >= 2)
            def _():
                wait_scatter(g - 2, b)

            def inner(gb, c):
                _pair_products(rows[b], outv[b], gb)
                return c

            lax.fori_loop(0, _G, inner, 0)
            start_scatter(g, b)
        return carry

    lax.fori_loop(0, ng // 2, outer, 0)
    wait_scatter(ng - 2, 0)
    wait_scatter(ng - 1, 1)


def kernel(X, v):
    B, F = X.shape
    Vn, F2, D = v.shape
    assert F == _FIELDS and F2 == _FIELDS and D == _EMBED
    assert B % (_NW * _G) == 0
    nb = B // _NW
    ng = nb // _G
    assert ng % 2 == 0

    x_flat = X.reshape(B * F).astype(jnp.int32)

    mesh = plsc.VectorSubcoreMesh(core_axis_name="c", subcore_axis_name="s")
    f32 = jnp.float32
    run = pl.kernel(
        functools.partial(_sc_body, nb, ng),
        mesh=mesh,
        compiler_params=pltpu.CompilerParams(use_tc_tiling_on_sc=False),
        out_type=jax.ShapeDtypeStruct((B, _NPAIRS, D), f32),
        scratch_types=[
            pltpu.VMEM((_G * _FIELDS,), jnp.int32),
            pltpu.VMEM((_G * _FIELDS,), jnp.int32),
            pltpu.VMEM((_G * _FIELDS, _FIELDS, _EMBED), f32),
            pltpu.VMEM((_G * _FIELDS, _FIELDS, _EMBED), f32),
            pltpu.VMEM((_G, _NPAIRS, _EMBED), f32),
            pltpu.VMEM((_G, _NPAIRS, _EMBED), f32),
            pltpu.SemaphoreType.DMA,
            pltpu.SemaphoreType.DMA,
            pltpu.SemaphoreType.DMA,
            pltpu.SemaphoreType.DMA,
        ],
    )
    return run(x_flat, v)


# speedup vs baseline: 1.4844x; 1.4844x over previous
import functools

import jax
import jax.numpy as jnp
import numpy as np
from jax import lax
from jax.experimental import pallas as pl
from jax.experimental.pallas import tpu as pltpu
from jax.experimental.pallas import tpu_sc as plsc

_FIELDS = 26
_EMBED = 16
_NPAIRS = (_FIELDS * (_FIELDS - 1)) // 2
_IU_R, _IU_C = np.triu_indices(_FIELDS, k=1)

_NC = 2
_NS = 16
_NW = _NC * _NS
_G = 4
_BC = 16
_GPC = _BC // _G
_PD = _NPAIRS * _EMBED


def _pairs_for_batch(rows_ref, ostage_ref, gb, lb_vec, iota16):
    rbase = gb * _FIELDS

    def body(p, carry):
        i, j = carry
        a = rows_ref[rbase + i, j, :]
        b = rows_ref[rbase + j, i, :]
        pd_vec = iota16 + p * _EMBED
        plsc.store_scatter(ostage_ref, [pd_vec, lb_vec], a * b)
        last = j == (_FIELDS - 1)
        i2 = jnp.where(last, i + 1, i)
        j2 = jnp.where(last, i + 2, j + 1)
        return (i2, j2)

    lax.fori_loop(0, _NPAIRS, body, (jnp.int32(0), jnp.int32(1)))


def _sc_body(nb, nchunk, x_hbm, v_hbm, out_hbm,
             idx_v, rows_v, ostage, gsem, osem):
    wid = lax.axis_index("s") * _NC + lax.axis_index("c")
    base = wid * nb
    iota16 = lax.iota(jnp.int32, _EMBED)

    def out_copy(c):
        return pltpu.make_async_copy(
            ostage, out_hbm.at[:, pl.ds((base + c * _BC), _BC)], osem)

    def chunk_body(c, carry):
        for lg in range(_GPC):
            g = c * _GPC + lg
            pltpu.sync_copy(
                x_hbm.at[pl.ds((base + g * _G) * _FIELDS, _G * _FIELDS)],
                idx_v)
            gather = pltpu.make_async_copy(v_hbm.at[idx_v], rows_v, gsem)
            gather.start()
            if lg == 0:
                @pl.when(c > 0)
                def _():
                    out_copy(c - 1).wait()
            gather.wait()

            def inner(gb, cc):
                lb_vec = jnp.broadcast_to(lg * _G + gb, (_EMBED,))
                _pairs_for_batch(rows_v, ostage, gb, lb_vec, iota16)
                return cc

            lax.fori_loop(0, _G, inner, 0)
        out_copy(c).start()
        return carry

    lax.fori_loop(0, nchunk, chunk_body, 0)
    out_copy(nchunk - 1).wait()


def kernel(X, v):
    B, F = X.shape
    Vn, F2, D = v.shape
    assert F == _FIELDS and F2 == _FIELDS and D == _EMBED
    assert B % (_NW * _BC) == 0
    nb = B // _NW
    nchunk = nb // _BC

    x_flat = X.reshape(B * F).astype(jnp.int32)

    mesh = plsc.VectorSubcoreMesh(core_axis_name="c", subcore_axis_name="s")
    f32 = jnp.float32
    run = pl.kernel(
        functools.partial(_sc_body, nb, nchunk),
        mesh=mesh,
        compiler_params=pltpu.CompilerParams(
            use_tc_tiling_on_sc=False, needs_layout_passes=False),
        out_type=jax.ShapeDtypeStruct((_PD, B), f32),
        scratch_types=[
            pltpu.VMEM((_G * _FIELDS,), jnp.int32),
            pltpu.VMEM((_G * _FIELDS, _FIELDS, _EMBED), f32),
            pltpu.VMEM((_PD, _BC), f32),
            pltpu.SemaphoreType.DMA,
            pltpu.SemaphoreType.DMA,
        ],
    )
    out2 = run(x_flat, v)
    return out2.reshape(_NPAIRS, _EMBED, B).transpose(2, 0, 1)


# speedup vs baseline: 1.6078x; 1.0831x over previous
import functools

import jax
import jax.numpy as jnp
import numpy as np
from jax import lax
from jax.experimental import pallas as pl
from jax.experimental.pallas import tpu as pltpu
from jax.experimental.pallas import tpu_sc as plsc

_FIELDS = 26
_EMBED = 16
_NPAIRS = (_FIELDS * (_FIELDS - 1)) // 2
_IU_R, _IU_C = np.triu_indices(_FIELDS, k=1)

_NC = 2
_NS = 16
_NW = _NC * _NS
_G = 4
_BC = 16
_GPC = _BC // _G
_PD = _NPAIRS * _EMBED


def _pairs_for_batch(rows_ref, ostage_ref, gb, lb_vec, iota16):
    rbase = gb * _FIELDS

    def body(p, carry):
        i, j = carry
        a = rows_ref[rbase + i, j, :]
        b = rows_ref[rbase + j, i, :]
        pd_vec = iota16 + p * _EMBED
        plsc.store_scatter(ostage_ref, [pd_vec, lb_vec], a * b)
        last = j == (_FIELDS - 1)
        i2 = jnp.where(last, i + 1, i)
        j2 = jnp.where(last, i + 2, j + 1)
        return (i2, j2)

    lax.fori_loop(0, _NPAIRS, body, (jnp.int32(0), jnp.int32(1)))


def _sc_body(nb, nchunk, x_hbm, v_hbm, out_hbm,
             idx_v, rows_v, ostage, gsem, osem):
    wid = lax.axis_index("s") * _NC + lax.axis_index("c")
    base = wid * nb
    iota16 = lax.iota(jnp.int32, _EMBED)

    def out_copy(c):
        return pltpu.make_async_copy(
            ostage, out_hbm.at[:, pl.ds((base + c * _BC), _BC)], osem)

    def chunk_body(c, carry):
        for lg in range(_GPC):
            g = c * _GPC + lg
            pltpu.sync_copy(
                x_hbm.at[pl.ds((base + g * _G) * _FIELDS, _G * _FIELDS)],
                idx_v)
            gather = pltpu.make_async_copy(v_hbm.at[idx_v], rows_v, gsem)
            gather.start()
            if lg == 0:
                @pl.when(c > 0)
                def _():
                    out_copy(c - 1).wait()
            gather.wait()

            def inner(gb, cc):
                lb_vec = jnp.broadcast_to(lg * _G + gb, (_EMBED,))
                _pairs_for_batch(rows_v, ostage, gb, lb_vec, iota16)
                return cc

            lax.fori_loop(0, _G, inner, 0)
        out_copy(c).start()
        return carry

    lax.fori_loop(0, nchunk, chunk_body, 0)
    out_copy(nchunk - 1).wait()


_TS = 32
_NSTRIP = 3125
_TPW = (_NSTRIP + _NW - 1) // _NW


def _t_body(vt_hbm, t2_hbm, stage0, stage1, trans0, trans1,
            isem0, isem1, osem0, osem1):
    stage = (stage0, stage1)
    trans = (trans0, trans1)
    isem = (isem0, isem1)
    osem = (osem0, osem1)
    wid = lax.axis_index("s") * _NC + lax.axis_index("c")
    iota16 = lax.iota(jnp.int32, _EMBED)

    def r0_of(t):
        return jnp.minimum(wid + t * _NW, _NSTRIP - 1) * _TS

    def in_copy(t, buf):
        return pltpu.make_async_copy(
            vt_hbm.at[:, :, pl.ds(r0_of(t), _TS)], stage[buf], isem[buf])

    def out_copy(t, buf):
        return pltpu.make_async_copy(
            trans[buf], t2_hbm.at[pl.ds(r0_of(t), _TS)], osem[buf])

    in_copy(0, 0).start()

    def outer(tt, carry):
        for b in (0, 1):
            t = tt * 2 + b

            @pl.when(t + 1 < _TPW)
            def _():
                in_copy(t + 1, (b + 1) % 2).start()

            in_copy(t, b).wait()

            @pl.when(t >= 2)
            def _():
                out_copy(t - 2, b).wait()

            def per_field(f, cc):
                for d in range(_EMBED):
                    for rc in range(_TS // _EMBED):
                        vals = stage[b][f, d, pl.ds(rc * _EMBED, _EMBED)]
                        plsc.store_scatter(
                            trans[b],
                            [iota16 + rc * _EMBED,
                             jnp.broadcast_to(f, (_EMBED,)),
                             jnp.broadcast_to(d, (_EMBED,))],
                            vals)
                return cc

            lax.fori_loop(0, _FIELDS, per_field, 0)
            out_copy(t, b).start()
        return carry

    lax.fori_loop(0, _TPW // 2, outer, 0)
    out_copy(_TPW - 2, 0).wait()
    out_copy(_TPW - 1, 1).wait()


def kernel(X, v):
    B, F = X.shape
    Vn, F2, D = v.shape
    assert F == _FIELDS and F2 == _FIELDS and D == _EMBED
    assert B % (_NW * _BC) == 0
    nb = B // _NW
    nchunk = nb // _BC

    x_flat = X.reshape(B * F).astype(jnp.int32)
    vt = jnp.transpose(v, (1, 2, 0))

    mesh = plsc.VectorSubcoreMesh(core_axis_name="c", subcore_axis_name="s")
    f32 = jnp.float32
    run_t = pl.kernel(
        _t_body,
        mesh=mesh,
        compiler_params=pltpu.CompilerParams(
            use_tc_tiling_on_sc=False, needs_layout_passes=False),
        out_type=jax.ShapeDtypeStruct((Vn, F, D), f32),
        scratch_types=[
            pltpu.VMEM((_FIELDS, _EMBED, _TS), f32),
            pltpu.VMEM((_FIELDS, _EMBED, _TS), f32),
            pltpu.VMEM((_TS, _FIELDS, _EMBED), f32),
            pltpu.VMEM((_TS, _FIELDS, _EMBED), f32),
            pltpu.SemaphoreType.DMA,
            pltpu.SemaphoreType.DMA,
            pltpu.SemaphoreType.DMA,
            pltpu.SemaphoreType.DMA,
        ],
    )
    run = pl.kernel(
        functools.partial(_sc_body, nb, nchunk),
        mesh=mesh,
        compiler_params=pltpu.CompilerParams(
            use_tc_tiling_on_sc=False, needs_layout_passes=False),
        out_type=jax.ShapeDtypeStruct((_PD, B), f32),
        scratch_types=[
            pltpu.VMEM((_G * _FIELDS,), jnp.int32),
            pltpu.VMEM((_G * _FIELDS, _FIELDS, _EMBED), f32),
            pltpu.VMEM((_PD, _BC), f32),
            pltpu.SemaphoreType.DMA,
            pltpu.SemaphoreType.DMA,
        ],
    )
    table2 = run_t(vt)
    out2 = run(x_flat, table2)
    return out2.reshape(_NPAIRS, _EMBED, B).transpose(2, 0, 1)
